# baseline (speedup 1.0000x reference)
import functools
import jax
import jax.numpy as jnp
from jax import lax
from jax.experimental import pallas as pl
from jax.experimental.pallas import tpu as pltpu
from jax.experimental.pallas import tpu_sc as plsc

NUM_SEGMENTS = 16
D_MODEL = 4096

_info = plsc.get_sparse_core_info()
_NC, _NS = _info.num_cores, _info.num_subcores
_NW = _NC * _NS
_L = 16

_B = 4 * 8192
_BPW = _B // _NW
_CH = 4
_NCHK = _BPW // _CH
_ROW = D_MODEL
_CHW = _CH * _ROW
_KU = 8


def _body(ids_hbm, table_hbm, out_hbm, idx_v, tab_v, buf_v, sem0, sem1):
    wid = lax.axis_index("s") * _NC + lax.axis_index("c")
    base = wid * _BPW * _ROW
    pltpu.sync_copy(ids_hbm.at[wid], idx_v)
    pltpu.sync_copy(table_hbm, tab_v)
    lanes = lax.iota(jnp.int32, _L)

    def assemble(j, p):
        grp = idx_v[pl.ds((j // 4) * _L, _L)]
        for q in range(_CH):
            lane = (j % 4) * _CH + q
            s = jnp.sum(jnp.where(lanes == lane, grp, 0))
            src0 = s * _ROW
            dst0 = p * _CHW + q * _ROW

            def cp(k, carry):
                for u in range(_KU):
                    o = (k * _KU + u) * _L
                    buf_v[pl.ds(dst0 + o, _L)] = tab_v[pl.ds(src0 + o, _L)]
                return carry

            lax.fori_loop(0, _ROW // (_KU * _L), cp, 0)

    def copy_out(j, p, sem):
        return pltpu.make_async_copy(
            buf_v.at[pl.ds(p * _CHW, _CHW)],
            out_hbm.at[pl.ds(base + j * _CHW, _CHW)],
            sem,
        )

    def step(j, carry):
        p = lax.rem(j, 2)
        @pl.when((j >= 2) & (p == 0))
        def _():
            copy_out(j - 2, 0, sem0).wait()
        @pl.when((j >= 2) & (p == 1))
        def _():
            copy_out(j - 2, 1, sem1).wait()
        assemble(j, p)
        @pl.when(p == 0)
        def _():
            copy_out(j, 0, sem0).start()
        @pl.when(p == 1)
        def _():
            copy_out(j, 1, sem1).start()
        return carry

    lax.fori_loop(0, _NCHK, step, 0)
    copy_out(_NCHK - 2, 0, sem0).wait()
    copy_out(_NCHK - 1, 1, sem1).wait()


def kernel(segment_ids, table):
    ids = segment_ids.reshape(_NW, _BPW).astype(jnp.int32)
    run = functools.partial(
        pl.kernel,
        mesh=plsc.VectorSubcoreMesh(core_axis_name="c", subcore_axis_name="s"),
        out_type=jax.ShapeDtypeStruct((_B * _ROW,), jnp.float32),
        compiler_params=pltpu.CompilerParams(needs_layout_passes=False),
        scratch_types=[
            pltpu.VMEM((_BPW,), jnp.int32),
            pltpu.VMEM((NUM_SEGMENTS * _ROW,), jnp.float32),
            pltpu.VMEM((2 * _CHW,), jnp.float32),
            pltpu.SemaphoreType.DMA,
            pltpu.SemaphoreType.DMA,
        ],
    )(_body)
    out = run(ids, table.reshape(-1))
    return out.reshape(segment_ids.shape[0], segment_ids.shape[1], D_MODEL)
